# Initial kernel scaffold; baseline (speedup 1.0000x reference)
import jax
import jax.numpy as jnp
from jax.experimental import pallas as pl


def kernel(x, edge_index, W1_l, b1, W1_r, W2_l, b2, W2_r):
    raise NotImplementedError("write your pallas kernel here")


# speedup vs baseline: 3.9876x; 3.9876x over previous
import functools

import jax
import jax.numpy as jnp
from jax import lax
from jax.experimental import pallas as pl
from jax.experimental.pallas import tpu as pltpu
from jax.experimental.pallas import tpu_sc as plsc

N_NODES = 10000
NP = 10240
GARBAGE = N_NODES
N_EDGES = 160000
NW = 32
EPW = 5120
NBATCH = 40
BATCH = 128
ROWS_PER_SUB = NP // 16
DCH = 128
BLK = 512
GRID = NP // BLK


def _sc_agg(values, srcp, dstp, zeros_h):
    mesh = plsc.VectorSubcoreMesh(core_axis_name="c", subcore_axis_name="s")

    @functools.partial(
        pl.kernel, mesh=mesh,
        out_type=jax.ShapeDtypeStruct((2, NP, DCH), jnp.float32),
        scratch_types=[
            pltpu.VMEM((NBATCH, BATCH), jnp.int32),
            pltpu.VMEM((NBATCH, BATCH), jnp.int32),
            pltpu.VMEM((BATCH, DCH), jnp.float32),
            pltpu.VMEM_SHARED((NP, DCH), jnp.float32),
        ],
    )
    def body(values_h, srcp_h, dstp_h, zeros_hr, out_h,
             src_v, dst_v, rows_v, acc_sh):
        cid = lax.axis_index("c")
        sid = lax.axis_index("s")
        wid = cid * 16 + sid
        rps = pl.ds(sid * ROWS_PER_SUB, ROWS_PER_SUB)
        pltpu.sync_copy(zeros_hr.at[rps], acc_sh.at[rps])
        pltpu.sync_copy(srcp_h.at[wid], src_v)
        pltpu.sync_copy(dstp_h.at[wid], dst_v)
        plsc.subcore_barrier()

        def batch(b, carry):
            pltpu.sync_copy(values_h.at[src_v.at[b]], rows_v)
            pltpu.sync_copy(rows_v, acc_sh.at[dst_v.at[b]], add=True)
            return carry
        lax.fori_loop(0, NBATCH, batch, 0)

        plsc.subcore_barrier()
        pltpu.sync_copy(acc_sh.at[rps], out_h.at[cid, rps])

    return body(values, srcp, dstp, zeros_h)


def _sc_count(dstp, ones_h, zc_h):
    mesh = plsc.VectorSubcoreMesh(core_axis_name="c", subcore_axis_name="s")

    @functools.partial(
        pl.kernel, mesh=mesh,
        out_type=jax.ShapeDtypeStruct((2, NP, DCH), jnp.float32),
        scratch_types=[
            pltpu.VMEM((NBATCH, BATCH), jnp.int32),
            pltpu.VMEM((BATCH, DCH), jnp.float32),
            pltpu.VMEM_SHARED((NP, DCH), jnp.float32),
        ],
    )
    def body(dstp_h, ones_hr, zc_hr, cnt_h, dst_v, ones_v, cacc_sh):
        cid = lax.axis_index("c")
        sid = lax.axis_index("s")
        wid = cid * 16 + sid
        rps = pl.ds(sid * ROWS_PER_SUB, ROWS_PER_SUB)
        pltpu.sync_copy(zc_hr.at[rps], cacc_sh.at[rps])
        pltpu.sync_copy(dstp_h.at[wid], dst_v)
        pltpu.sync_copy(ones_hr, ones_v)
        plsc.subcore_barrier()

        def batch(b, carry):
            pltpu.sync_copy(ones_v, cacc_sh.at[dst_v.at[b]], add=True)
            return carry
        lax.fori_loop(0, NBATCH, batch, 0)

        plsc.subcore_barrier()
        pltpu.sync_copy(cacc_sh.at[rps], cnt_h.at[cid, rps])

    return body(dstp, ones_h, zc_h)


def _tc_mm1(a0, a1, cntp, x, W1_l, b1, W1_r, W2_l, b2, W2_r):
    def kfn(a0_r, a1_r, cnt_r, x_r, w1l_r, b1_r, w1r_r, w2l_r, b2_r, w2r_r,
            p_r, q_r):
        cnt16 = jnp.sum(cnt_r[0] + cnt_r[1], axis=1, keepdims=True)
        inv = float(DCH) / jnp.maximum(cnt16, float(DCH))
        m0 = (a0_r[0] + a0_r[1]) * inv
        m1 = (a1_r[0] + a1_r[1]) * inv
        w1l = w1l_r[...]
        h = (jnp.dot(m0, w1l[:DCH], preferred_element_type=jnp.float32)
             + jnp.dot(m1, w1l[DCH:], preferred_element_type=jnp.float32)
             + jnp.dot(x_r[...], w1r_r[...],
                       preferred_element_type=jnp.float32)
             + b1_r[...])
        h = jnp.maximum(h, 0.0)
        p_r[...] = jnp.dot(h, w2l_r[...], preferred_element_type=jnp.float32)
        q_r[...] = (jnp.dot(h, w2r_r[...], preferred_element_type=jnp.float32)
                    + b2_r[...])

    grid = (GRID,)
    return pl.pallas_call(
        kfn,
        grid=grid,
        in_specs=[
            pl.BlockSpec((2, BLK, DCH), lambda i: (0, i, 0)),
            pl.BlockSpec((2, BLK, DCH), lambda i: (0, i, 0)),
            pl.BlockSpec((2, BLK, DCH), lambda i: (0, i, 0)),
            pl.BlockSpec((BLK, 256), lambda i: (i, 0)),
            pl.BlockSpec((256, 512), lambda i: (0, 0)),
            pl.BlockSpec((1, 512), lambda i: (0, 0)),
            pl.BlockSpec((256, 512), lambda i: (0, 0)),
            pl.BlockSpec((512, 128), lambda i: (0, 0)),
            pl.BlockSpec((1, 128), lambda i: (0, 0)),
            pl.BlockSpec((512, 128), lambda i: (0, 0)),
        ],
        out_specs=[
            pl.BlockSpec((BLK, 128), lambda i: (i, 0)),
            pl.BlockSpec((BLK, 128), lambda i: (i, 0)),
        ],
        out_shape=[
            jax.ShapeDtypeStruct((NP, 128), jnp.float32),
            jax.ShapeDtypeStruct((NP, 128), jnp.float32),
        ],
    )(a0, a1, cntp, x, W1_l, b1, W1_r, W2_l, b2, W2_r)


def _tc_final(a2, cntp, q):
    def kfn(a_r, cnt_r, q_r, o_r):
        cnt16 = jnp.sum(cnt_r[0] + cnt_r[1], axis=1, keepdims=True)
        inv = float(DCH) / jnp.maximum(cnt16, float(DCH))
        s = (a_r[0] + a_r[1]) * inv + q_r[...]
        m = jnp.max(s, axis=-1, keepdims=True)
        e = jnp.exp(s - m)
        o_r[...] = s - m - jnp.log(jnp.sum(e, axis=-1, keepdims=True))

    return pl.pallas_call(
        kfn,
        grid=(GRID,),
        in_specs=[
            pl.BlockSpec((2, BLK, 128), lambda i: (0, i, 0)),
            pl.BlockSpec((2, BLK, DCH), lambda i: (0, i, 0)),
            pl.BlockSpec((BLK, 128), lambda i: (i, 0)),
        ],
        out_specs=pl.BlockSpec((BLK, 128), lambda i: (i, 0)),
        out_shape=jax.ShapeDtypeStruct((NP, 128), jnp.float32),
    )(a2, cntp, q)


def kernel(x, edge_index, W1_l, b1, W1_r, W2_l, b2, W2_r):
    src = edge_index[0].astype(jnp.int32)
    dst = edge_index[1].astype(jnp.int32)
    pad = EPW - N_EDGES // NW
    srcp = jnp.concatenate(
        [src.reshape(NW, N_EDGES // NW),
         jnp.zeros((NW, pad), jnp.int32)], axis=1)
    dstp = jnp.concatenate(
        [dst.reshape(NW, N_EDGES // NW),
         jnp.full((NW, pad), GARBAGE, jnp.int32)], axis=1)
    srcp = srcp.reshape(NW, NBATCH, BATCH)
    dstp3 = dstp.reshape(NW, NBATCH, BATCH)

    x_pad = jnp.zeros((NP, 256), jnp.float32).at[:N_NODES].set(x)
    zeros_h = jnp.zeros((NP, DCH), jnp.float32)
    ones_h = jnp.ones((BATCH, DCH), jnp.float32)

    cntp = _sc_count(dstp3, ones_h, zeros_h)
    a0 = _sc_agg(x_pad[:, :DCH], srcp, dstp3, zeros_h)
    a1 = _sc_agg(x_pad[:, DCH:], srcp, dstp3, zeros_h)
    p, q = _tc_mm1(a0, a1, cntp, x_pad, W1_l, b1.reshape(1, 512), W1_r,
                   W2_l, b2.reshape(1, 128), W2_r)
    a2 = _sc_agg(p, srcp, dstp3, zeros_h)
    out = _tc_final(a2, cntp, q)
    return out[:N_NODES]


# speedup vs baseline: 4.2525x; 1.0664x over previous
import functools

import jax
import jax.numpy as jnp
from jax import lax
from jax.experimental import pallas as pl
from jax.experimental.pallas import tpu as pltpu
from jax.experimental.pallas import tpu_sc as plsc

N_NODES = 10000
NP = 10240
GARBAGE = N_NODES
N_EDGES = 160000
NW = 32
EPW = 5120
NBATCH = 40
BATCH = 128
NBUF = 2
ROWS_PER_SUB = NP // 16
DCH = 128
BLK = 512
GRID = NP // BLK


def _sc_agg(values, srcp, dstp, zeros_h):
    mesh = plsc.VectorSubcoreMesh(core_axis_name="c", subcore_axis_name="s")

    @functools.partial(
        pl.kernel, mesh=mesh,
        out_type=jax.ShapeDtypeStruct((2, NP, DCH), jnp.float32),
        scratch_types=[
            pltpu.VMEM((NBATCH, BATCH), jnp.int32),
            pltpu.VMEM((NBATCH, BATCH), jnp.int32),
            pltpu.VMEM((NBUF, BATCH, DCH), jnp.float32),
            pltpu.VMEM_SHARED((NP, DCH), jnp.float32),
            pltpu.SemaphoreType.DMA((NBUF,)),
            pltpu.SemaphoreType.DMA((NBUF,)),
        ],
    )
    def body(values_h, srcp_h, dstp_h, zeros_hr, out_h,
             src_v, dst_v, rows_v, acc_sh, gsem, ssem):
        cid = lax.axis_index("c")
        sid = lax.axis_index("s")
        wid = cid * 16 + sid
        rps = pl.ds(sid * ROWS_PER_SUB, ROWS_PER_SUB)
        pltpu.sync_copy(zeros_hr.at[rps], acc_sh.at[rps])
        pltpu.sync_copy(srcp_h.at[wid], src_v)
        pltpu.sync_copy(dstp_h.at[wid], dst_v)
        plsc.subcore_barrier()

        def gather(b, j):
            pltpu.make_async_copy(values_h.at[src_v.at[b]], rows_v.at[j],
                                  gsem.at[j]).start()

        def scatter(b, j):
            pltpu.make_async_copy(rows_v.at[j], acc_sh.at[dst_v.at[b]],
                                  ssem.at[j]).start(add=True)

        for j in range(NBUF):
            gather(j, j)

        def group(g, carry):
            for j in range(NBUF):
                b = g * NBUF + j
                pltpu.make_async_copy(values_h.at[src_v.at[b]],
                                      rows_v.at[j], gsem.at[j]).wait()
                scatter(b, j)
            for j in range(NBUF):
                b = g * NBUF + j
                pltpu.make_async_copy(rows_v.at[j], acc_sh.at[dst_v.at[b]],
                                      ssem.at[j]).wait()
                b2 = b + NBUF

                @pl.when(b2 < NBATCH)
                def _():
                    gather(b2, j)
            return carry
        lax.fori_loop(0, NBATCH // NBUF, group, 0)

        plsc.subcore_barrier()
        pltpu.sync_copy(acc_sh.at[rps], out_h.at[cid, rps])

    return body(values, srcp, dstp, zeros_h)


def _sc_count(dstp, ones_h, zc_h):
    mesh = plsc.VectorSubcoreMesh(core_axis_name="c", subcore_axis_name="s")

    @functools.partial(
        pl.kernel, mesh=mesh,
        out_type=jax.ShapeDtypeStruct((2, NP, DCH), jnp.float32),
        scratch_types=[
            pltpu.VMEM((NBATCH, BATCH), jnp.int32),
            pltpu.VMEM((BATCH, DCH), jnp.float32),
            pltpu.VMEM_SHARED((NP, DCH), jnp.float32),
            pltpu.SemaphoreType.DMA,
        ],
    )
    def body(dstp_h, ones_hr, zc_hr, cnt_h, dst_v, ones_v, cacc_sh, sem):
        cid = lax.axis_index("c")
        sid = lax.axis_index("s")
        wid = cid * 16 + sid
        rps = pl.ds(sid * ROWS_PER_SUB, ROWS_PER_SUB)
        pltpu.sync_copy(zc_hr.at[rps], cacc_sh.at[rps])
        pltpu.sync_copy(dstp_h.at[wid], dst_v)
        pltpu.sync_copy(ones_hr, ones_v)
        plsc.subcore_barrier()

        def batch(b, carry):
            pltpu.make_async_copy(ones_v, cacc_sh.at[dst_v.at[b]],
                                  sem).start(add=True)
            return carry
        lax.fori_loop(0, NBATCH, batch, 0)

        def drain(b, carry):
            pltpu.make_async_copy(ones_v, cacc_sh.at[dst_v.at[b]],
                                  sem).wait()
            return carry
        lax.fori_loop(0, NBATCH, drain, 0)

        plsc.subcore_barrier()
        pltpu.sync_copy(cacc_sh.at[rps], cnt_h.at[cid, rps])

    return body(dstp, ones_h, zc_h)


def _tc_mm1(a0, a1, cntp, x, W1_l, b1, W1_r, W2_l, b2, W2_r):
    def kfn(a0_r, a1_r, cnt_r, x_r, w1l_r, b1_r, w1r_r, w2l_r, b2_r, w2r_r,
            p_r, q_r):
        cnt16 = jnp.sum(cnt_r[0] + cnt_r[1], axis=1, keepdims=True)
        inv = float(DCH) / jnp.maximum(cnt16, float(DCH))
        m0 = (a0_r[0] + a0_r[1]) * inv
        m1 = (a1_r[0] + a1_r[1]) * inv
        w1l = w1l_r[...]
        h = (jnp.dot(m0, w1l[:DCH], preferred_element_type=jnp.float32)
             + jnp.dot(m1, w1l[DCH:], preferred_element_type=jnp.float32)
             + jnp.dot(x_r[...], w1r_r[...],
                       preferred_element_type=jnp.float32)
             + b1_r[...])
        h = jnp.maximum(h, 0.0)
        p_r[...] = jnp.dot(h, w2l_r[...], preferred_element_type=jnp.float32)
        q_r[...] = (jnp.dot(h, w2r_r[...], preferred_element_type=jnp.float32)
                    + b2_r[...])

    grid = (GRID,)
    return pl.pallas_call(
        kfn,
        grid=grid,
        in_specs=[
            pl.BlockSpec((2, BLK, DCH), lambda i: (0, i, 0)),
            pl.BlockSpec((2, BLK, DCH), lambda i: (0, i, 0)),
            pl.BlockSpec((2, BLK, DCH), lambda i: (0, i, 0)),
            pl.BlockSpec((BLK, 256), lambda i: (i, 0)),
            pl.BlockSpec((256, 512), lambda i: (0, 0)),
            pl.BlockSpec((1, 512), lambda i: (0, 0)),
            pl.BlockSpec((256, 512), lambda i: (0, 0)),
            pl.BlockSpec((512, 128), lambda i: (0, 0)),
            pl.BlockSpec((1, 128), lambda i: (0, 0)),
            pl.BlockSpec((512, 128), lambda i: (0, 0)),
        ],
        out_specs=[
            pl.BlockSpec((BLK, 128), lambda i: (i, 0)),
            pl.BlockSpec((BLK, 128), lambda i: (i, 0)),
        ],
        out_shape=[
            jax.ShapeDtypeStruct((NP, 128), jnp.float32),
            jax.ShapeDtypeStruct((NP, 128), jnp.float32),
        ],
    )(a0, a1, cntp, x, W1_l, b1, W1_r, W2_l, b2, W2_r)


def _tc_final(a2, cntp, q):
    def kfn(a_r, cnt_r, q_r, o_r):
        cnt16 = jnp.sum(cnt_r[0] + cnt_r[1], axis=1, keepdims=True)
        inv = float(DCH) / jnp.maximum(cnt16, float(DCH))
        s = (a_r[0] + a_r[1]) * inv + q_r[...]
        m = jnp.max(s, axis=-1, keepdims=True)
        e = jnp.exp(s - m)
        o_r[...] = s - m - jnp.log(jnp.sum(e, axis=-1, keepdims=True))

    return pl.pallas_call(
        kfn,
        grid=(GRID,),
        in_specs=[
            pl.BlockSpec((2, BLK, 128), lambda i: (0, i, 0)),
            pl.BlockSpec((2, BLK, DCH), lambda i: (0, i, 0)),
            pl.BlockSpec((BLK, 128), lambda i: (i, 0)),
        ],
        out_specs=pl.BlockSpec((BLK, 128), lambda i: (i, 0)),
        out_shape=jax.ShapeDtypeStruct((NP, 128), jnp.float32),
    )(a2, cntp, q)


def kernel(x, edge_index, W1_l, b1, W1_r, W2_l, b2, W2_r):
    src = edge_index[0].astype(jnp.int32)
    dst = edge_index[1].astype(jnp.int32)
    pad = EPW - N_EDGES // NW
    srcp = jnp.concatenate(
        [src.reshape(NW, N_EDGES // NW),
         jnp.zeros((NW, pad), jnp.int32)], axis=1)
    dstp = jnp.concatenate(
        [dst.reshape(NW, N_EDGES // NW),
         jnp.full((NW, pad), GARBAGE, jnp.int32)], axis=1)
    srcp = srcp.reshape(NW, NBATCH, BATCH)
    dstp3 = dstp.reshape(NW, NBATCH, BATCH)

    x_pad = jnp.zeros((NP, 256), jnp.float32).at[:N_NODES].set(x)
    zeros_h = jnp.zeros((NP, DCH), jnp.float32)
    ones_h = jnp.ones((BATCH, DCH), jnp.float32)

    cntp = _sc_count(dstp3, ones_h, zeros_h)
    a0 = _sc_agg(x_pad[:, :DCH], srcp, dstp3, zeros_h)
    a1 = _sc_agg(x_pad[:, DCH:], srcp, dstp3, zeros_h)
    p, q = _tc_mm1(a0, a1, cntp, x_pad, W1_l, b1.reshape(1, 512), W1_r,
                   W2_l, b2.reshape(1, 128), W2_r)
    a2 = _sc_agg(p, srcp, dstp3, zeros_h)
    out = _tc_final(a2, cntp, q)
    return out[:N_NODES]
